# baseline (speedup 1.0000x reference)
import functools

import jax
import jax.numpy as jnp
from jax import lax
from jax.experimental import pallas as pl
from jax.experimental.pallas import tpu as pltpu
from jax.experimental.pallas import tpu_sc as plsc

N_V = 10000
N_E = 5000
NNZ = 320000
D = 128
ND8 = D // 16

NC, NS, L = 2, 16, 16
NW = NC * NS
CHUNK = NNZ // NW
BATCH = 128
NFULL = CHUNK // BATCH
TAIL = CHUNK - NFULL * BATCH

HV_PAD = 10240
HE_PAD = 5120
N_EP = 5120
N_VP = 10240

_mesh = plsc.VectorSubcoreMesh(core_axis_name="c", subcore_axis_name="s")


def _wid():
    return lax.axis_index("s") * NC + lax.axis_index("c")


def _zeros16():
    return jnp.zeros((L,), jnp.float32)


def _fill_zbuf(zbuf):
    z = _zeros16()
    for r in range(L):
        for t in range(ND8):
            zbuf[r, pl.ds(16 * t, 16)] = z


def _stage_batch(dst, src, base, width=BATCH):
    for t in range(width // 16):
        dst[pl.ds(16 * t, 16)] = src[pl.ds(base + 16 * t, 16)]


@functools.partial(
    pl.kernel,
    out_type=(
        jax.ShapeDtypeStruct((NC * HV_PAD,), jnp.float32),
        jax.ShapeDtypeStruct((NC * HE_PAD,), jnp.float32),
    ),
    mesh=_mesh,
    scratch_types=[
        pltpu.VMEM((CHUNK,), jnp.int32),
        pltpu.VMEM((CHUNK,), jnp.int32),
        pltpu.VMEM((BATCH,), jnp.int32),
        pltpu.VMEM((BATCH,), jnp.int32),
        pltpu.VMEM((16,), jnp.int32),
        pltpu.VMEM((16,), jnp.int32),
        pltpu.VMEM((BATCH,), jnp.float32),
        pltpu.VMEM((640,), jnp.float32),
        pltpu.VMEM_SHARED((HV_PAD,), jnp.float32),
        pltpu.VMEM_SHARED((HE_PAD,), jnp.float32),
    ],
)
def _degrees(nidx, eidx, degv_out, dege_out,
             nchunk, echunk, nbuf, ebuf, ntail, etail, ones, zb, hv, he):
    c = lax.axis_index("c")
    s = lax.axis_index("s")
    wid = _wid()

    one = jnp.ones((16,), jnp.float32)
    z = _zeros16()
    for t in range(BATCH // 16):
        ones[pl.ds(16 * t, 16)] = one
    for t in range(640 // 16):
        zb[pl.ds(16 * t, 16)] = z

    pltpu.sync_copy(zb, hv.at[pl.ds(640 * s, 640)])
    pltpu.sync_copy(zb.at[pl.ds(0, 320)], he.at[pl.ds(320 * s, 320)])
    plsc.subcore_barrier()

    base0 = wid * CHUNK
    pltpu.sync_copy(nidx.at[pl.ds(base0, CHUNK)], nchunk)
    pltpu.sync_copy(eidx.at[pl.ds(base0, CHUNK)], echunk)

    def body(j, carry):
        base = j * BATCH
        _stage_batch(nbuf, nchunk, base)
        _stage_batch(ebuf, echunk, base)
        pltpu.sync_copy(ones, hv.at[nbuf], add=True)
        pltpu.sync_copy(ones, he.at[ebuf], add=True)
        return carry

    lax.fori_loop(0, NFULL, body, 0)

    tb = NFULL * BATCH
    ntail[pl.ds(0, 16)] = nchunk[pl.ds(tb, 16)]
    etail[pl.ds(0, 16)] = echunk[pl.ds(tb, 16)]
    pltpu.sync_copy(ones.at[pl.ds(0, 16)], hv.at[ntail], add=True)
    pltpu.sync_copy(ones.at[pl.ds(0, 16)], he.at[etail], add=True)

    plsc.subcore_barrier()
    pltpu.sync_copy(hv.at[pl.ds(640 * s, 640)], zb)
    pltpu.sync_copy(zb, degv_out.at[pl.ds(c * HV_PAD + 640 * s, 640)])
    pltpu.sync_copy(he.at[pl.ds(320 * s, 320)], zb.at[pl.ds(0, 320)])
    pltpu.sync_copy(zb.at[pl.ds(0, 320)],
                    dege_out.at[pl.ds(c * HE_PAD + 320 * s, 320)])


BLK = 64
NBLK_V = N_V // BLK


def _scale_rows(rows, d0, d1, nrows, out=None):
    dst = rows if out is None else out
    for k4 in range(nrows // 16):
        dv = d0[pl.ds(16 * k4, 16)] + d1[pl.ds(16 * k4, 16)]
        inv = jnp.where(dv > 0, 1.0 / dv, 0.0)
        for r in range(16):
            sv = jnp.broadcast_to(inv[r], (16,))
            row = 16 * k4 + r
            for t in range(ND8):
                dst[row, pl.ds(16 * t, 16)] = rows[row, pl.ds(16 * t, 16)] * sv


@functools.partial(
    pl.kernel,
    out_type=jax.ShapeDtypeStruct((N_V, D), jnp.float32),
    mesh=_mesh,
    scratch_types=[
        pltpu.VMEM((BLK, D), jnp.float32),
        pltpu.VMEM((BLK,), jnp.float32),
        pltpu.VMEM((BLK,), jnp.float32),
        pltpu.SemaphoreType.DMA,
    ],
)
def _normalize_x(x, degv_part, xnorm, rows, d0, d1, sem):
    wid = _wid()
    trip = (NBLK_V + 1 - wid + NW - 1) // NW

    def body(k, carry):
        g = wid + NW * k
        r0 = BLK * g

        @pl.when(g < NBLK_V)
        def _full():
            a = pltpu.async_copy(x.at[pl.ds(r0, BLK)], rows, sem)
            b = pltpu.async_copy(degv_part.at[pl.ds(r0, BLK)], d0, sem)
            cc = pltpu.async_copy(degv_part.at[pl.ds(HV_PAD + r0, BLK)], d1, sem)
            a.wait()
            b.wait()
            cc.wait()
            _scale_rows(rows, d0, d1, BLK)
            pltpu.sync_copy(rows, xnorm.at[pl.ds(r0, BLK)])

        @pl.when(g == NBLK_V)
        def _tail():
            rt = rows.at[pl.ds(0, 16)]
            d0t = d0.at[pl.ds(0, 16)]
            d1t = d1.at[pl.ds(0, 16)]
            pltpu.sync_copy(x.at[pl.ds(r0, 16)], rt)
            pltpu.sync_copy(degv_part.at[pl.ds(r0, 16)], d0t)
            pltpu.sync_copy(degv_part.at[pl.ds(HV_PAD + r0, 16)], d1t)
            _scale_rows(rows, d0, d1, 16)
            pltpu.sync_copy(rt, xnorm.at[pl.ds(r0, 16)])

        return carry

    lax.fori_loop(0, trip, body, 0)


AB = 128
NFULL_AG = CHUNK // AB
NB2 = NFULL_AG // 2
CHUNK_PAD = (NFULL_AG + 1) * AB


def _make_aggregate(n_out_pad):
    rows_per_tile = n_out_pad // NS
    nfull16 = rows_per_tile // 16
    assert rows_per_tile % 16 == 0

    @functools.partial(
        pl.kernel,
        out_type=jax.ShapeDtypeStruct((NC, n_out_pad, D), jnp.float32),
        mesh=_mesh,
        scratch_types=[
            pltpu.VMEM((CHUNK_PAD,), jnp.int32),
            pltpu.VMEM((AB,), jnp.int32),
            pltpu.VMEM((AB,), jnp.int32),
            pltpu.VMEM((AB,), jnp.int32),
            pltpu.VMEM((AB,), jnp.int32),
            pltpu.VMEM((16,), jnp.int32),
            pltpu.VMEM((AB, D), jnp.float32),
            pltpu.VMEM((AB, D), jnp.float32),
            pltpu.VMEM((16, D), jnp.float32),
            pltpu.VMEM_SHARED((n_out_pad, D), jnp.float32),
            pltpu.SemaphoreType.DMA,
            pltpu.SemaphoreType.DMA,
        ],
    )
    def agg(table, gidx, sidx, part_out,
            gchunk, gbA, sbA, gbB, sbB, stail, rowsA, rowsB, zb, acc,
            sem, sem2):
        c = lax.axis_index("c")
        s = lax.axis_index("s")
        wid = _wid()

        _fill_zbuf(zb)
        r0 = rows_per_tile * s
        for k in range(nfull16):
            pltpu.sync_copy(zb, acc.at[pl.ds(r0 + 16 * k, 16)])
        plsc.subcore_barrier()

        base0 = wid * CHUNK
        pltpu.sync_copy(gidx.at[pl.ds(base0, CHUNK)], gchunk.at[pl.ds(0, CHUNK)])
        zi = jnp.zeros((16,), jnp.int32)
        for t in range((CHUNK_PAD - CHUNK) // 16):
            gchunk[pl.ds(CHUNK + 16 * t, 16)] = zi

        _stage_batch(gbA, gchunk, 0, AB)
        pltpu.async_copy(table.at[gbA], rowsA, sem)
        pltpu.async_copy(sidx.at[pl.ds(base0, AB)], sbA, sem2)

        def body(i, carry):
            baseB = (2 * i + 1) * AB
            _stage_batch(gbB, gchunk, baseB, AB)
            pltpu.make_async_copy(table.at[gbA], rowsA, sem).wait()
            pltpu.async_copy(table.at[gbB], rowsB, sem)
            pltpu.make_async_copy(sidx.at[pl.ds(base0, AB)], sbA, sem2).wait()
            pltpu.async_copy(sidx.at[pl.ds(base0 + baseB, AB)], sbB, sem2)
            pltpu.sync_copy(rowsA, acc.at[sbA], add=True)

            baseA = (2 * i + 2) * AB
            _stage_batch(gbA, gchunk, baseA, AB)
            pltpu.make_async_copy(table.at[gbB], rowsB, sem).wait()
            pltpu.async_copy(table.at[gbA], rowsA, sem)
            pltpu.make_async_copy(sidx.at[pl.ds(base0, AB)], sbB, sem2).wait()
            pltpu.async_copy(sidx.at[pl.ds(base0 + baseA, AB)], sbA, sem2)
            pltpu.sync_copy(rowsB, acc.at[sbB], add=True)
            return carry

        lax.fori_loop(0, NB2, body, 0)

        pltpu.make_async_copy(table.at[gbA], rowsA, sem).wait()
        pltpu.make_async_copy(sidx.at[pl.ds(base0, AB)], sbA, sem2).wait()
        stail[pl.ds(0, 16)] = sbA[pl.ds(0, 16)]
        pltpu.sync_copy(rowsA.at[pl.ds(0, 16)], acc.at[stail], add=True)

        plsc.subcore_barrier()
        st = rowsA.at[pl.ds(0, 64)]
        for k in range(rows_per_tile // 64):
            pltpu.sync_copy(acc.at[pl.ds(r0 + 64 * k, 64)], st)
            pltpu.sync_copy(st, part_out.at[c, pl.ds(r0 + 64 * k, 64)])

    return agg


_v2e = _make_aggregate(N_EP)
_e2v = _make_aggregate(N_VP)


@functools.partial(
    pl.kernel,
    out_type=(
        jax.ShapeDtypeStruct((N_E, D), jnp.float32),
        jax.ShapeDtypeStruct((N_EP, D), jnp.float32),
    ),
    mesh=_mesh,
    scratch_types=[
        pltpu.VMEM((BLK, D), jnp.float32),
        pltpu.VMEM((BLK, D), jnp.float32),
        pltpu.VMEM((BLK,), jnp.float32),
        pltpu.VMEM((BLK,), jnp.float32),
        pltpu.SemaphoreType.DMA,
    ],
)
def _combine_edges(ef_part, dege_part, edge_feat, efn, ba, bb, d0, d1, sem):
    wid = _wid()
    nblk = N_EP // BLK
    trip = (nblk - wid + NW - 1) // NW

    def _sum_rows(nrows):
        for r in range(nrows):
            for t in range(ND8):
                ba[r, pl.ds(16 * t, 16)] = (ba[r, pl.ds(16 * t, 16)]
                                            + bb[r, pl.ds(16 * t, 16)])

    def body(k, carry):
        g = wid + NW * k
        r0 = BLK * g
        a = pltpu.async_copy(ef_part.at[0, pl.ds(r0, BLK)], ba, sem)
        b = pltpu.async_copy(ef_part.at[1, pl.ds(r0, BLK)], bb, sem)
        cc = pltpu.async_copy(dege_part.at[pl.ds(r0, BLK)], d0, sem)
        dd = pltpu.async_copy(dege_part.at[pl.ds(HE_PAD + r0, BLK)], d1, sem)
        a.wait()
        b.wait()
        cc.wait()
        dd.wait()
        _sum_rows(BLK)

        @pl.when(g < 78)
        def _full():
            pltpu.sync_copy(ba, edge_feat.at[pl.ds(r0, BLK)])

        @pl.when(g == 78)
        def _ragged():
            pltpu.sync_copy(ba.at[pl.ds(0, 8)], edge_feat.at[pl.ds(r0, 8)])

        _scale_rows(ba, d0, d1, BLK, out=bb)
        pltpu.sync_copy(bb, efn.at[pl.ds(r0, BLK)])
        return carry

    lax.fori_loop(0, trip, body, 0)


@functools.partial(
    pl.kernel,
    out_type=jax.ShapeDtypeStruct((N_V, D), jnp.float32),
    mesh=_mesh,
    scratch_types=[
        pltpu.VMEM((BLK, D), jnp.float32),
        pltpu.VMEM((BLK, D), jnp.float32),
        pltpu.SemaphoreType.DMA,
    ],
)
def _combine_nodes(nf_part, node_feat, ba, bb, sem):
    wid = _wid()
    trip = (NBLK_V + 1 - wid + NW - 1) // NW

    def _sum_rows(nrows):
        for r in range(nrows):
            for t in range(ND8):
                ba[r, pl.ds(16 * t, 16)] = (ba[r, pl.ds(16 * t, 16)]
                                            + bb[r, pl.ds(16 * t, 16)])

    def body(k, carry):
        g = wid + NW * k
        r0 = BLK * g

        @pl.when(g < NBLK_V)
        def _full():
            a = pltpu.async_copy(nf_part.at[0, pl.ds(r0, BLK)], ba, sem)
            b = pltpu.async_copy(nf_part.at[1, pl.ds(r0, BLK)], bb, sem)
            a.wait()
            b.wait()
            _sum_rows(BLK)
            pltpu.sync_copy(ba, node_feat.at[pl.ds(r0, BLK)])

        @pl.when(g == NBLK_V)
        def _tail():
            bat = ba.at[pl.ds(0, 16)]
            bbt = bb.at[pl.ds(0, 16)]
            pltpu.sync_copy(nf_part.at[0, pl.ds(r0, 16)], bat)
            pltpu.sync_copy(nf_part.at[1, pl.ds(r0, 16)], bbt)
            _sum_rows(16)
            pltpu.sync_copy(bat, node_feat.at[pl.ds(r0, 16)])

        return carry

    lax.fori_loop(0, trip, body, 0)


def kernel(X, node_idx, edge_idx):
    node_idx = node_idx.astype(jnp.int32)
    edge_idx = edge_idx.astype(jnp.int32)
    pad = jnp.zeros((AB,), jnp.int32)
    nidx_p = jnp.concatenate([node_idx, pad])
    eidx_p = jnp.concatenate([edge_idx, pad])
    degv_part, dege_part = _degrees(node_idx, edge_idx)
    xnorm = _normalize_x(X, degv_part)
    ef_part = _v2e(xnorm, node_idx, eidx_p)
    edge_feat, efn = _combine_edges(ef_part, dege_part)
    nf_part = _e2v(efn, edge_idx, nidx_p)
    node_feat = _combine_nodes(nf_part)
    return (node_feat, edge_feat)


# speedup vs baseline: 1.3335x; 1.3335x over previous
import functools

import jax
import jax.numpy as jnp
from jax import lax
from jax.experimental import pallas as pl
from jax.experimental.pallas import tpu as pltpu
from jax.experimental.pallas import tpu_sc as plsc

N_V = 10000
N_E = 5000
NNZ = 320000
D = 128
ND8 = D // 16

NC, NS, L = 2, 16, 16
NW = NC * NS
CHUNK = NNZ // NW
BATCH = 128
NFULL = CHUNK // BATCH
TAIL = CHUNK - NFULL * BATCH

HV_PAD = 10240
HE_PAD = 5120
N_EP = 5120
N_VP = 10240

_mesh = plsc.VectorSubcoreMesh(core_axis_name="c", subcore_axis_name="s")


def _wid():
    return lax.axis_index("s") * NC + lax.axis_index("c")


def _zeros16():
    return jnp.zeros((L,), jnp.float32)


def _fill_zbuf(zbuf):
    z = _zeros16()
    for r in range(L):
        for t in range(ND8):
            zbuf[r, pl.ds(16 * t, 16)] = z


def _stage_batch(dst, src, base, width=BATCH):
    for t in range(width // 16):
        dst[pl.ds(16 * t, 16)] = src[pl.ds(base + 16 * t, 16)]


@functools.partial(
    pl.kernel,
    out_type=(
        jax.ShapeDtypeStruct((NC * HV_PAD,), jnp.float32),
        jax.ShapeDtypeStruct((NC * HE_PAD,), jnp.float32),
    ),
    mesh=_mesh,
    scratch_types=[
        pltpu.VMEM((CHUNK,), jnp.int32),
        pltpu.VMEM((CHUNK,), jnp.int32),
        pltpu.VMEM((BATCH,), jnp.int32),
        pltpu.VMEM((BATCH,), jnp.int32),
        pltpu.VMEM((16,), jnp.int32),
        pltpu.VMEM((16,), jnp.int32),
        pltpu.VMEM((BATCH,), jnp.float32),
        pltpu.VMEM((640,), jnp.float32),
        pltpu.VMEM_SHARED((HV_PAD,), jnp.float32),
        pltpu.VMEM_SHARED((HE_PAD,), jnp.float32),
    ],
)
def _degrees(nidx, eidx, degv_out, dege_out,
             nchunk, echunk, nbuf, ebuf, ntail, etail, ones, zb, hv, he):
    c = lax.axis_index("c")
    s = lax.axis_index("s")
    wid = _wid()

    one = jnp.ones((16,), jnp.float32)
    z = _zeros16()
    for t in range(BATCH // 16):
        ones[pl.ds(16 * t, 16)] = one
    for t in range(640 // 16):
        zb[pl.ds(16 * t, 16)] = z

    pltpu.sync_copy(zb, hv.at[pl.ds(640 * s, 640)])
    pltpu.sync_copy(zb.at[pl.ds(0, 320)], he.at[pl.ds(320 * s, 320)])
    plsc.subcore_barrier()

    base0 = wid * CHUNK
    pltpu.sync_copy(nidx.at[pl.ds(base0, CHUNK)], nchunk)
    pltpu.sync_copy(eidx.at[pl.ds(base0, CHUNK)], echunk)

    def body(j, carry):
        base = j * BATCH
        _stage_batch(nbuf, nchunk, base)
        _stage_batch(ebuf, echunk, base)
        pltpu.sync_copy(ones, hv.at[nbuf], add=True)
        pltpu.sync_copy(ones, he.at[ebuf], add=True)
        return carry

    lax.fori_loop(0, NFULL, body, 0)

    tb = NFULL * BATCH
    ntail[pl.ds(0, 16)] = nchunk[pl.ds(tb, 16)]
    etail[pl.ds(0, 16)] = echunk[pl.ds(tb, 16)]
    pltpu.sync_copy(ones.at[pl.ds(0, 16)], hv.at[ntail], add=True)
    pltpu.sync_copy(ones.at[pl.ds(0, 16)], he.at[etail], add=True)

    plsc.subcore_barrier()
    pltpu.sync_copy(hv.at[pl.ds(640 * s, 640)], zb)
    pltpu.sync_copy(zb, degv_out.at[pl.ds(c * HV_PAD + 640 * s, 640)])
    pltpu.sync_copy(he.at[pl.ds(320 * s, 320)], zb.at[pl.ds(0, 320)])
    pltpu.sync_copy(zb.at[pl.ds(0, 320)],
                    dege_out.at[pl.ds(c * HE_PAD + 320 * s, 320)])


BLK = 64
NBLK_V = N_V // BLK


def _scale_rows(rows, d0, d1, nrows, out=None):
    dst = rows if out is None else out
    for k4 in range(nrows // 16):
        dv = d0[pl.ds(16 * k4, 16)] + d1[pl.ds(16 * k4, 16)]
        inv = jnp.where(dv > 0, 1.0 / dv, 0.0)
        for r in range(16):
            sv = jnp.broadcast_to(inv[r], (16,))
            row = 16 * k4 + r
            for t in range(ND8):
                dst[row, pl.ds(16 * t, 16)] = rows[row, pl.ds(16 * t, 16)] * sv


@functools.partial(
    pl.kernel,
    out_type=jax.ShapeDtypeStruct((N_V, D), jnp.float32),
    mesh=_mesh,
    scratch_types=[
        pltpu.VMEM((BLK, D), jnp.float32),
        pltpu.VMEM((BLK,), jnp.float32),
        pltpu.VMEM((BLK,), jnp.float32),
        pltpu.SemaphoreType.DMA,
    ],
)
def _normalize_x(x, degv_part, xnorm, rows, d0, d1, sem):
    wid = _wid()
    trip = (NBLK_V + 1 - wid + NW - 1) // NW

    def body(k, carry):
        g = wid + NW * k
        r0 = BLK * g

        @pl.when(g < NBLK_V)
        def _full():
            a = pltpu.async_copy(x.at[pl.ds(r0, BLK)], rows, sem)
            b = pltpu.async_copy(degv_part.at[pl.ds(r0, BLK)], d0, sem)
            cc = pltpu.async_copy(degv_part.at[pl.ds(HV_PAD + r0, BLK)], d1, sem)
            a.wait()
            b.wait()
            cc.wait()
            _scale_rows(rows, d0, d1, BLK)
            pltpu.sync_copy(rows, xnorm.at[pl.ds(r0, BLK)])

        @pl.when(g == NBLK_V)
        def _tail():
            rt = rows.at[pl.ds(0, 16)]
            d0t = d0.at[pl.ds(0, 16)]
            d1t = d1.at[pl.ds(0, 16)]
            pltpu.sync_copy(x.at[pl.ds(r0, 16)], rt)
            pltpu.sync_copy(degv_part.at[pl.ds(r0, 16)], d0t)
            pltpu.sync_copy(degv_part.at[pl.ds(HV_PAD + r0, 16)], d1t)
            _scale_rows(rows, d0, d1, 16)
            pltpu.sync_copy(rt, xnorm.at[pl.ds(r0, 16)])

        return carry

    lax.fori_loop(0, trip, body, 0)


def _make_aggregate(n_out_pad):
    rows_per_tile = n_out_pad // NS
    nfull16 = rows_per_tile // 16
    assert rows_per_tile % 16 == 0

    @functools.partial(
        pl.kernel,
        out_type=jax.ShapeDtypeStruct((NC, n_out_pad, D), jnp.float32),
        mesh=_mesh,
        scratch_types=[
            pltpu.VMEM((CHUNK,), jnp.int32),
            pltpu.VMEM((CHUNK,), jnp.int32),
            pltpu.VMEM((BATCH,), jnp.int32),
            pltpu.VMEM((BATCH,), jnp.int32),
            pltpu.VMEM((16,), jnp.int32),
            pltpu.VMEM((16,), jnp.int32),
            pltpu.VMEM((BATCH, D), jnp.float32),
            pltpu.VMEM((16, D), jnp.float32),
            pltpu.VMEM_SHARED((n_out_pad, D), jnp.float32),
            pltpu.SemaphoreType.DMA,
        ],
    )
    def agg(table, gidx, sidx, part_out,
            gchunk, schunk, gbuf, sbuf, gtail, stail, rows, zb, acc, sem):
        c = lax.axis_index("c")
        s = lax.axis_index("s")
        wid = _wid()

        _fill_zbuf(zb)
        r0 = rows_per_tile * s
        for k in range(nfull16):
            pltpu.sync_copy(zb, acc.at[pl.ds(r0 + 16 * k, 16)])
        plsc.subcore_barrier()

        base0 = wid * CHUNK
        pltpu.sync_copy(gidx.at[pl.ds(base0, CHUNK)], gchunk)
        pltpu.sync_copy(sidx.at[pl.ds(base0, CHUNK)], schunk)

        def body(j, carry):
            base = j * BATCH
            _stage_batch(gbuf, gchunk, base)
            _stage_batch(sbuf, schunk, base)
            pltpu.async_copy(table.at[gbuf], rows, sem).wait()
            pltpu.sync_copy(rows, acc.at[sbuf], add=True)
            return carry

        lax.fori_loop(0, NFULL, body, 0)

        tb = NFULL * BATCH
        gtail[pl.ds(0, 16)] = gchunk[pl.ds(tb, 16)]
        stail[pl.ds(0, 16)] = schunk[pl.ds(tb, 16)]
        pltpu.async_copy(table.at[gtail], zb, sem).wait()
        pltpu.sync_copy(zb, acc.at[stail], add=True)

        plsc.subcore_barrier()
        st = rows.at[pl.ds(0, 64)]
        for k in range(rows_per_tile // 64):
            pltpu.sync_copy(acc.at[pl.ds(r0 + 64 * k, 64)], st)
            pltpu.sync_copy(st, part_out.at[c, pl.ds(r0 + 64 * k, 64)])

    return agg


_v2e = _make_aggregate(N_EP)
_e2v = _make_aggregate(N_VP)


@functools.partial(
    pl.kernel,
    out_type=(
        jax.ShapeDtypeStruct((N_E, D), jnp.float32),
        jax.ShapeDtypeStruct((N_EP, D), jnp.float32),
    ),
    mesh=_mesh,
    scratch_types=[
        pltpu.VMEM((BLK, D), jnp.float32),
        pltpu.VMEM((BLK, D), jnp.float32),
        pltpu.VMEM((BLK,), jnp.float32),
        pltpu.VMEM((BLK,), jnp.float32),
        pltpu.SemaphoreType.DMA,
    ],
)
def _combine_edges(ef_part, dege_part, edge_feat, efn, ba, bb, d0, d1, sem):
    wid = _wid()
    nblk = N_EP // BLK
    trip = (nblk - wid + NW - 1) // NW

    def _sum_rows(nrows):
        for r in range(nrows):
            for t in range(ND8):
                ba[r, pl.ds(16 * t, 16)] = (ba[r, pl.ds(16 * t, 16)]
                                            + bb[r, pl.ds(16 * t, 16)])

    def body(k, carry):
        g = wid + NW * k
        r0 = BLK * g
        a = pltpu.async_copy(ef_part.at[0, pl.ds(r0, BLK)], ba, sem)
        b = pltpu.async_copy(ef_part.at[1, pl.ds(r0, BLK)], bb, sem)
        cc = pltpu.async_copy(dege_part.at[pl.ds(r0, BLK)], d0, sem)
        dd = pltpu.async_copy(dege_part.at[pl.ds(HE_PAD + r0, BLK)], d1, sem)
        a.wait()
        b.wait()
        cc.wait()
        dd.wait()
        _sum_rows(BLK)

        @pl.when(g < 78)
        def _full():
            pltpu.sync_copy(ba, edge_feat.at[pl.ds(r0, BLK)])

        @pl.when(g == 78)
        def _ragged():
            pltpu.sync_copy(ba.at[pl.ds(0, 8)], edge_feat.at[pl.ds(r0, 8)])

        _scale_rows(ba, d0, d1, BLK, out=bb)
        pltpu.sync_copy(bb, efn.at[pl.ds(r0, BLK)])
        return carry

    lax.fori_loop(0, trip, body, 0)


@functools.partial(
    pl.kernel,
    out_type=jax.ShapeDtypeStruct((N_V, D), jnp.float32),
    mesh=_mesh,
    scratch_types=[
        pltpu.VMEM((BLK, D), jnp.float32),
        pltpu.VMEM((BLK, D), jnp.float32),
        pltpu.SemaphoreType.DMA,
    ],
)
def _combine_nodes(nf_part, node_feat, ba, bb, sem):
    wid = _wid()
    trip = (NBLK_V + 1 - wid + NW - 1) // NW

    def _sum_rows(nrows):
        for r in range(nrows):
            for t in range(ND8):
                ba[r, pl.ds(16 * t, 16)] = (ba[r, pl.ds(16 * t, 16)]
                                            + bb[r, pl.ds(16 * t, 16)])

    def body(k, carry):
        g = wid + NW * k
        r0 = BLK * g

        @pl.when(g < NBLK_V)
        def _full():
            a = pltpu.async_copy(nf_part.at[0, pl.ds(r0, BLK)], ba, sem)
            b = pltpu.async_copy(nf_part.at[1, pl.ds(r0, BLK)], bb, sem)
            a.wait()
            b.wait()
            _sum_rows(BLK)
            pltpu.sync_copy(ba, node_feat.at[pl.ds(r0, BLK)])

        @pl.when(g == NBLK_V)
        def _tail():
            bat = ba.at[pl.ds(0, 16)]
            bbt = bb.at[pl.ds(0, 16)]
            pltpu.sync_copy(nf_part.at[0, pl.ds(r0, 16)], bat)
            pltpu.sync_copy(nf_part.at[1, pl.ds(r0, 16)], bbt)
            _sum_rows(16)
            pltpu.sync_copy(bat, node_feat.at[pl.ds(r0, 16)])

        return carry

    lax.fori_loop(0, trip, body, 0)


def kernel(X, node_idx, edge_idx):
    node_idx = node_idx.astype(jnp.int32)
    edge_idx = edge_idx.astype(jnp.int32)
    degv_part, dege_part = _degrees(node_idx, edge_idx)
    xnorm = _normalize_x(X, degv_part)
    ef_part = _v2e(xnorm, node_idx, edge_idx)
    edge_feat, efn = _combine_edges(ef_part, dege_part)
    nf_part = _e2v(efn, edge_idx, node_idx)
    node_feat = _combine_nodes(nf_part)
    return (node_feat, edge_feat)
